# Initial kernel scaffold; baseline (speedup 1.0000x reference)
import jax
import jax.numpy as jnp
from jax.experimental import pallas as pl


def kernel(x, edge_index, edge_attr, W1, b1, W2, b2, Wc, bc, Wm, bm, Ws, bs, high, deterministic):
    raise NotImplementedError("write your pallas kernel here")


# speedup vs baseline: 3.8498x; 3.8498x over previous
import functools

import jax
import jax.numpy as jnp
from jax import lax
from jax.experimental import pallas as pl
from jax.experimental.pallas import tpu as pltpu
from jax.experimental.pallas import tpu_sc as plsc

_N = 10000
_E = 320000
_NODE = 128
_HID = 32
_CH = 128
_NCH = _E // _CH
_NW = 32
_CPW = (_NCH + _NW - 1) // _NW
_NT = 16
_RPT = 624
_REM = _N - _NT * _RPT


def _sc_edge(src_hbm, dst_hbm, a_hbm, p_hbm, q_hbm, zs_hbm, zd_hbm, ones_hbm,
             s_out, d_out,
             sS, sD, src_v, dst_v, a_v, p_v, q_v, ones_v, zbuf, dbuf,
             sem0, sem1):
    cid = lax.axis_index("c")
    sid = lax.axis_index("s")
    wid = sid * 2 + cid
    tb = sid * _RPT

    pltpu.sync_copy(zs_hbm.at[pl.ds(tb, _RPT)], zbuf)
    pltpu.sync_copy(zbuf, sS.at[pl.ds(tb, _RPT)])
    pltpu.sync_copy(zd_hbm.at[pl.ds(tb, _RPT)], dbuf)
    pltpu.sync_copy(dbuf, sD.at[pl.ds(tb, _RPT)])
    pltpu.sync_copy(ones_hbm, ones_v)

    @pl.when(sid == _NT - 1)
    def _():
        pltpu.sync_copy(zs_hbm.at[pl.ds(_NT * _RPT, _REM)],
                        zbuf.at[pl.ds(0, _REM)])
        pltpu.sync_copy(zbuf.at[pl.ds(0, _REM)],
                        sS.at[pl.ds(_NT * _RPT, _REM)])
        pltpu.sync_copy(zd_hbm.at[pl.ds(_NT * _RPT, _REM)],
                        dbuf.at[pl.ds(0, _REM)])
        pltpu.sync_copy(dbuf.at[pl.ds(0, _REM)],
                        sD.at[pl.ds(_NT * _RPT, _REM)])

    plsc.subcore_barrier()

    def _chunk(i, carry):
        c = wid + i * _NW

        @pl.when(c < _NCH)
        def _():
            base = c * _CH
            pltpu.sync_copy(src_hbm.at[pl.ds(base, _CH)], src_v)
            pltpu.sync_copy(dst_hbm.at[pl.ds(base, _CH)], dst_v)
            pltpu.sync_copy(a_hbm.at[pl.ds(base, _CH)], a_v)
            gp = pltpu.async_copy(p_hbm.at[src_v], p_v, sem0)
            gq = pltpu.async_copy(q_hbm.at[dst_v], q_v, sem1)
            gp.wait()
            gq.wait()
            for r in range(_CH):
                a_v[r, pl.ds(0, 16)] = jnp.maximum(
                    a_v[r, pl.ds(0, 16)] + p_v[r, pl.ds(0, 16)]
                    + q_v[r, pl.ds(0, 16)], 0.0)
                a_v[r, pl.ds(16, 16)] = jnp.maximum(
                    a_v[r, pl.ds(16, 16)] + p_v[r, pl.ds(16, 16)]
                    + q_v[r, pl.ds(16, 16)], 0.0)
            pltpu.sync_copy(a_v, sS.at[src_v], add=True)
            pltpu.sync_copy(ones_v, sD.at[src_v], add=True)

        return carry

    lax.fori_loop(0, _CPW, _chunk, 0)
    plsc.subcore_barrier()

    pltpu.sync_copy(sS.at[pl.ds(tb, _RPT)], zbuf)
    pltpu.sync_copy(zbuf, s_out.at[cid, pl.ds(tb, _RPT)])
    pltpu.sync_copy(sD.at[pl.ds(tb, _RPT)], dbuf)
    pltpu.sync_copy(dbuf, d_out.at[cid, pl.ds(tb, _RPT)])

    @pl.when(sid == _NT - 1)
    def _():
        pltpu.sync_copy(sS.at[pl.ds(_NT * _RPT, _REM)],
                        zbuf.at[pl.ds(0, _REM)])
        pltpu.sync_copy(zbuf.at[pl.ds(0, _REM)],
                        s_out.at[cid, pl.ds(_NT * _RPT, _REM)])
        pltpu.sync_copy(sD.at[pl.ds(_NT * _RPT, _REM)],
                        dbuf.at[pl.ds(0, _REM)])
        pltpu.sync_copy(dbuf.at[pl.ds(0, _REM)],
                        d_out.at[cid, pl.ds(_NT * _RPT, _REM)])


def _pq_body(x_ref, wa_ref, wb_ref, p_ref, q_ref):
    xv = x_ref[...]
    p_ref[...] = jnp.dot(xv, wa_ref[...], preferred_element_type=jnp.float32)
    q_ref[...] = jnp.dot(xv, wb_ref[...], preferred_element_type=jnp.float32)


def _a_body(e_ref, w_ref, b_ref, o_ref):
    o_ref[...] = (jnp.dot(e_ref[...], w_ref[...],
                          preferred_element_type=jnp.float32) + b_ref[...])


def _head_body(x_ref, sp_ref, dp_ref, w2_ref, b2_ref,
               wcx_ref, wca_ref, wmx_ref, wma_ref, wsx_ref, wsa_ref,
               bc_ref, bm_ref, bs_ref, hi_ref, inv_ref, ord_ref):
    s = sp_ref[0] + sp_ref[1]
    deg = dp_ref[0] + dp_ref[1]
    agg = (jnp.dot(s, w2_ref[...], preferred_element_type=jnp.float32)
           + deg * b2_ref[...])
    xv = x_ref[...]
    cx = (jnp.dot(xv, wcx_ref[...], preferred_element_type=jnp.float32)
          + jnp.dot(agg, wca_ref[...], preferred_element_type=jnp.float32)
          + bc_ref[...] + 1e-10)
    conc = jax.nn.softplus(cx)
    inv_ref[...] = conc / (jnp.sum(conc) + 1e-20)
    am = (jnp.dot(xv, wmx_ref[...], preferred_element_type=jnp.float32)
          + jnp.dot(agg, wma_ref[...], preferred_element_type=jnp.float32)
          + bm_ref[...])
    av = (jnp.dot(xv, wsx_ref[...], preferred_element_type=jnp.float32)
          + jnp.dot(agg, wsa_ref[...], preferred_element_type=jnp.float32)
          + bs_ref[...])
    alpha = jax.nn.softplus(am + 1e-20) + 1e-20
    beta = jax.nn.softplus(av + 1e-20) + 1.0
    ord_ref[...] = alpha / (alpha + beta) * hi_ref[0, 0]


def kernel(x, edge_index, edge_attr, W1, b1, W2, b2, Wc, bc, Wm, bm, Ws, bs,
           high, deterministic):
    f32 = jnp.float32
    src = edge_index[0]
    dst = edge_index[1]
    w1a = W1[:_NODE]
    w1b = W1[_NODE:2 * _NODE]
    w1c = W1[2 * _NODE:]

    p_arr, q_arr = pl.pallas_call(
        _pq_body,
        out_shape=[jax.ShapeDtypeStruct((_N, _HID), f32)] * 2,
    )(x, w1a, w1b)

    eb = _E // 25
    a_arr = pl.pallas_call(
        _a_body,
        grid=(25,),
        in_specs=[pl.BlockSpec((eb, 16), lambda i: (i, 0)),
                  pl.BlockSpec((16, _HID), lambda i: (0, 0)),
                  pl.BlockSpec((1, _HID), lambda i: (0, 0))],
        out_specs=pl.BlockSpec((eb, _HID), lambda i: (i, 0)),
        out_shape=jax.ShapeDtypeStruct((_E, _HID), f32),
    )(edge_attr, w1c, b1.reshape(1, _HID))

    zs = jnp.zeros((_N, _HID), f32)
    zd = jnp.zeros((_N,), f32)
    ones = jnp.ones((_CH,), f32)

    sc_fn = pl.kernel(
        _sc_edge,
        out_type=[jax.ShapeDtypeStruct((2, _N, _HID), f32),
                  jax.ShapeDtypeStruct((2, _N), f32)],
        mesh=plsc.VectorSubcoreMesh(core_axis_name="c", subcore_axis_name="s"),
        compiler_params=pltpu.CompilerParams(use_tc_tiling_on_sc=False),
        scratch_types=[
            pltpu.VMEM_SHARED((_N, _HID), f32),
            pltpu.VMEM_SHARED((_N,), f32),
            pltpu.VMEM((_CH,), jnp.int32),
            pltpu.VMEM((_CH,), jnp.int32),
            pltpu.VMEM((_CH, _HID), f32),
            pltpu.VMEM((_CH, _HID), f32),
            pltpu.VMEM((_CH, _HID), f32),
            pltpu.VMEM((_CH,), f32),
            pltpu.VMEM((_RPT, _HID), f32),
            pltpu.VMEM((_RPT,), f32),
            pltpu.SemaphoreType.DMA,
            pltpu.SemaphoreType.DMA,
        ],
    )
    s_part, d_part = sc_fn(src, dst, a_arr, p_arr, q_arr, zs, zd, ones)
    d_part = d_part.reshape(2, _N, 1)

    inv, ordv = pl.pallas_call(
        _head_body,
        out_shape=[jax.ShapeDtypeStruct((_N, 1), f32)] * 2,
    )(x, s_part, d_part, W2, b2.reshape(1, _HID),
      Wc[:_NODE], Wc[_NODE:], Wm[:_NODE], Wm[_NODE:], Ws[:_NODE], Ws[_NODE:],
      bc.reshape(1, 1), bm.reshape(1, 1), bs.reshape(1, 1),
      jnp.reshape(high, (1, 1)).astype(f32))

    inventory_act = inv.reshape(100, 100)
    order_act = ordv.reshape(100, 100)[:, -10:].reshape(-1)
    return (inventory_act, order_act)


# speedup vs baseline: 5.2288x; 1.3582x over previous
import functools

import jax
import jax.numpy as jnp
from jax import lax
from jax.experimental import pallas as pl
from jax.experimental.pallas import tpu as pltpu
from jax.experimental.pallas import tpu_sc as plsc

_N = 10000
_E = 320000
_NODE = 128
_HID = 32
_CH = 128
_NW = 32
_EPW = _E // _NW
_NFC = _EPW // _CH
_TAIL = _EPW - _NFC * _CH
_NBUF = 4
_NT = 16
_RPT = 624
_REM = _N - _NT * _RPT


def _sc_edge(eidx_hbm, a_hbm, p_hbm, q_hbm, zs_hbm, zd_hbm, ones_hbm,
             s_out, d_out,
             sS, sD,
             ei0, ei1, ei2, ei3, a0, a1, a2, a3, p0, p1, p2, p3,
             q0, q1, q2, q3, ones_v, zbuf, dbuf, tidx,
             st0, st1, st2, st3, sg0, sg1, sg2, sg3, ss0, ss1, ss2, ss3):
    eiv = [ei0, ei1, ei2, ei3]
    av = [a0, a1, a2, a3]
    pv = [p0, p1, p2, p3]
    qv = [q0, q1, q2, q3]
    sst = [st0, st1, st2, st3]
    ssg = [sg0, sg1, sg2, sg3]
    sss = [ss0, ss1, ss2, ss3]
    cid = lax.axis_index("c")
    sid = lax.axis_index("s")
    wid = sid * 2 + cid
    wbase = wid * _EPW
    tb = sid * _RPT

    pltpu.sync_copy(zs_hbm.at[pl.ds(tb, _RPT)], zbuf)
    pltpu.sync_copy(zbuf, sS.at[pl.ds(tb, _RPT)])
    pltpu.sync_copy(zd_hbm.at[pl.ds(tb, _RPT)], dbuf)
    pltpu.sync_copy(dbuf, sD.at[pl.ds(tb, _RPT)])
    pltpu.sync_copy(ones_hbm, ones_v)

    @pl.when(sid == _NT - 1)
    def _():
        pltpu.sync_copy(zs_hbm.at[pl.ds(_NT * _RPT, _REM)],
                        zbuf.at[pl.ds(0, _REM)])
        pltpu.sync_copy(zbuf.at[pl.ds(0, _REM)],
                        sS.at[pl.ds(_NT * _RPT, _REM)])
        pltpu.sync_copy(zd_hbm.at[pl.ds(_NT * _RPT, _REM)],
                        dbuf.at[pl.ds(0, _REM)])
        pltpu.sync_copy(dbuf.at[pl.ds(0, _REM)],
                        sD.at[pl.ds(_NT * _RPT, _REM)])

    plsc.subcore_barrier()

    def stage(ci, b):
        base = wbase + ci * _CH
        pltpu.async_copy(eidx_hbm.at[:, pl.ds(base, _CH)], eiv[b], sst[b])
        pltpu.async_copy(a_hbm.at[pl.ds(base, _CH)], av[b], sst[b])

    def wait_stage(b):
        pltpu.make_async_copy(eidx_hbm.at[:, pl.ds(0, _CH)], eiv[b],
                              sst[b]).wait()
        pltpu.make_async_copy(a_hbm.at[pl.ds(0, _CH)], av[b], sst[b]).wait()

    def gather(b):
        pltpu.async_copy(p_hbm.at[eiv[b].at[0]], pv[b], ssg[b])
        pltpu.async_copy(q_hbm.at[eiv[b].at[1]], qv[b], ssg[b])

    def wait_gather(b):
        pltpu.make_async_copy(p_hbm.at[eiv[b].at[0]], pv[b], ssg[b]).wait()
        pltpu.make_async_copy(q_hbm.at[eiv[b].at[1]], qv[b], ssg[b]).wait()

    def compute(b, nrows):
        def _row(r, carry):
            av[b][r, pl.ds(0, 16)] = jnp.maximum(
                av[b][r, pl.ds(0, 16)] + pv[b][r, pl.ds(0, 16)]
                + qv[b][r, pl.ds(0, 16)], 0.0)
            av[b][r, pl.ds(16, 16)] = jnp.maximum(
                av[b][r, pl.ds(16, 16)] + pv[b][r, pl.ds(16, 16)]
                + qv[b][r, pl.ds(16, 16)], 0.0)
            return carry

        lax.fori_loop(0, nrows, _row, 0, unroll=8)

    def scatter(b):
        pltpu.async_copy(av[b], sS.at[eiv[b].at[0]], sss[b], add=True)
        pltpu.async_copy(ones_v, sD.at[eiv[b].at[0]], sss[b], add=True)

    def wait_scatter(b):
        pltpu.make_async_copy(av[b], sS.at[eiv[b].at[0]], sss[b]).wait()
        pltpu.make_async_copy(ones_v, sD.at[eiv[b].at[0]], sss[b]).wait()

    stage(0, 0)
    stage(1, 1)
    stage(2, 2)
    wait_stage(0)
    gather(0)

    _MAIN = _NFC - 2 * _NBUF + 2

    def _main(g, carry):
        for r in range(_NBUF):
            i = g * _NBUF + r
            wait_stage((r + 1) % _NBUF)
            gather((r + 1) % _NBUF)
            if r == 0:
                @pl.when(g > 0)
                def _():
                    wait_scatter((r + 3) % _NBUF)
            else:
                wait_scatter((r + 3) % _NBUF)
            stage(i + 3, (r + 3) % _NBUF)
            wait_gather(r)
            compute(r, _CH)
            scatter(r)
        return carry

    lax.fori_loop(0, _MAIN // _NBUF, _main, 0)

    for i in range(_MAIN, _NFC):
        b = i % _NBUF
        if i + 1 < _NFC:
            wait_stage((i + 1) % _NBUF)
            gather((i + 1) % _NBUF)
        wait_scatter((i - 1) % _NBUF)
        if i + 3 < _NFC:
            stage(i + 3, (i + 3) % _NBUF)
        wait_gather(b)
        compute(b, _CH)
        scatter(b)
    wait_scatter((_NFC - 1) % _NBUF)

    tbase = wbase + _NFC * _CH
    pltpu.sync_copy(eidx_hbm.at[:, pl.ds(tbase, _TAIL)], tidx)
    pltpu.sync_copy(a_hbm.at[pl.ds(tbase, _TAIL)], av[0].at[pl.ds(0, _TAIL)])
    gp = pltpu.async_copy(p_hbm.at[tidx.at[0]], pv[0].at[pl.ds(0, _TAIL)],
                          sst[0])
    gq = pltpu.async_copy(q_hbm.at[tidx.at[1]], qv[0].at[pl.ds(0, _TAIL)],
                          sst[1])
    gp.wait()
    gq.wait()
    compute(0, _TAIL)
    pltpu.sync_copy(av[0].at[pl.ds(0, _TAIL)], sS.at[tidx.at[0]], add=True)
    pltpu.sync_copy(ones_v.at[pl.ds(0, _TAIL)], sD.at[tidx.at[0]], add=True)

    plsc.subcore_barrier()

    pltpu.sync_copy(sS.at[pl.ds(tb, _RPT)], zbuf)
    pltpu.sync_copy(zbuf, s_out.at[cid, pl.ds(tb, _RPT)])
    pltpu.sync_copy(sD.at[pl.ds(tb, _RPT)], dbuf)
    pltpu.sync_copy(dbuf, d_out.at[cid, pl.ds(tb, _RPT)])

    @pl.when(sid == _NT - 1)
    def _():
        pltpu.sync_copy(sS.at[pl.ds(_NT * _RPT, _REM)],
                        zbuf.at[pl.ds(0, _REM)])
        pltpu.sync_copy(zbuf.at[pl.ds(0, _REM)],
                        s_out.at[cid, pl.ds(_NT * _RPT, _REM)])
        pltpu.sync_copy(sD.at[pl.ds(_NT * _RPT, _REM)],
                        dbuf.at[pl.ds(0, _REM)])
        pltpu.sync_copy(dbuf.at[pl.ds(0, _REM)],
                        d_out.at[cid, pl.ds(_NT * _RPT, _REM)])


def _pq_body(x_ref, wa_ref, wb_ref, p_ref, q_ref):
    xv = x_ref[...]
    p_ref[...] = jnp.dot(xv, wa_ref[...], preferred_element_type=jnp.float32)
    q_ref[...] = jnp.dot(xv, wb_ref[...], preferred_element_type=jnp.float32)


def _a_body(e_ref, w_ref, b_ref, o_ref):
    o_ref[...] = (jnp.dot(e_ref[...], w_ref[...],
                          preferred_element_type=jnp.float32) + b_ref[...])


def _head_body(x_ref, sp_ref, dp_ref, w2_ref, b2_ref,
               wcx_ref, wca_ref, wmx_ref, wma_ref, wsx_ref, wsa_ref,
               bc_ref, bm_ref, bs_ref, hi_ref, inv_ref, ord_ref):
    s = sp_ref[0] + sp_ref[1]
    deg = dp_ref[0] + dp_ref[1]
    agg = (jnp.dot(s, w2_ref[...], preferred_element_type=jnp.float32)
           + deg * b2_ref[...])
    xv = x_ref[...]
    cx = (jnp.dot(xv, wcx_ref[...], preferred_element_type=jnp.float32)
          + jnp.dot(agg, wca_ref[...], preferred_element_type=jnp.float32)
          + bc_ref[...] + 1e-10)
    conc = jax.nn.softplus(cx)
    inv_ref[...] = conc / (jnp.sum(conc) + 1e-20)
    am = (jnp.dot(xv, wmx_ref[...], preferred_element_type=jnp.float32)
          + jnp.dot(agg, wma_ref[...], preferred_element_type=jnp.float32)
          + bm_ref[...])
    av = (jnp.dot(xv, wsx_ref[...], preferred_element_type=jnp.float32)
          + jnp.dot(agg, wsa_ref[...], preferred_element_type=jnp.float32)
          + bs_ref[...])
    alpha = jax.nn.softplus(am + 1e-20) + 1e-20
    beta = jax.nn.softplus(av + 1e-20) + 1.0
    ord_ref[...] = alpha / (alpha + beta) * hi_ref[0, 0]


def kernel(x, edge_index, edge_attr, W1, b1, W2, b2, Wc, bc, Wm, bm, Ws, bs,
           high, deterministic):
    f32 = jnp.float32
    w1a = W1[:_NODE]
    w1b = W1[_NODE:2 * _NODE]
    w1c = W1[2 * _NODE:]

    p_arr, q_arr = pl.pallas_call(
        _pq_body,
        out_shape=[jax.ShapeDtypeStruct((_N, _HID), f32)] * 2,
    )(x, w1a, w1b)

    eb = _E // 25
    a_arr = pl.pallas_call(
        _a_body,
        grid=(25,),
        in_specs=[pl.BlockSpec((eb, 16), lambda i: (i, 0)),
                  pl.BlockSpec((16, _HID), lambda i: (0, 0)),
                  pl.BlockSpec((1, _HID), lambda i: (0, 0))],
        out_specs=pl.BlockSpec((eb, _HID), lambda i: (i, 0)),
        out_shape=jax.ShapeDtypeStruct((_E, _HID), f32),
    )(edge_attr, w1c, b1.reshape(1, _HID))

    zs = jnp.zeros((_N, _HID), f32)
    zd = jnp.zeros((_N,), f32)
    ones = jnp.ones((_CH,), f32)

    sc_fn = pl.kernel(
        _sc_edge,
        out_type=[jax.ShapeDtypeStruct((2, _N, _HID), f32),
                  jax.ShapeDtypeStruct((2, _N), f32)],
        mesh=plsc.VectorSubcoreMesh(core_axis_name="c", subcore_axis_name="s"),
        compiler_params=pltpu.CompilerParams(use_tc_tiling_on_sc=False),
        scratch_types=(
            [pltpu.VMEM_SHARED((_N, _HID), f32),
             pltpu.VMEM_SHARED((_N,), f32)]
            + [pltpu.VMEM((2, _CH), jnp.int32)] * _NBUF
            + [pltpu.VMEM((_CH, _HID), f32)] * (3 * _NBUF)
            + [pltpu.VMEM((_CH,), f32),
               pltpu.VMEM((_RPT, _HID), f32),
               pltpu.VMEM((_RPT,), f32),
               pltpu.VMEM((2, _TAIL), jnp.int32)]
            + [pltpu.SemaphoreType.DMA] * (3 * _NBUF)
        ),
    )
    s_part, d_part = sc_fn(edge_index, a_arr, p_arr, q_arr, zs, zd, ones)
    d_part = d_part.reshape(2, _N, 1)

    inv, ordv = pl.pallas_call(
        _head_body,
        out_shape=[jax.ShapeDtypeStruct((_N, 1), f32)] * 2,
    )(x, s_part, d_part, W2, b2.reshape(1, _HID),
      Wc[:_NODE], Wc[_NODE:], Wm[:_NODE], Wm[_NODE:], Ws[:_NODE], Ws[_NODE:],
      bc.reshape(1, 1), bm.reshape(1, 1), bs.reshape(1, 1),
      jnp.reshape(high, (1, 1)).astype(f32))

    inventory_act = inv.reshape(100, 100)
    order_act = ordv.reshape(100, 100)[:, -10:].reshape(-1)
    return (inventory_act, order_act)


# speedup vs baseline: 9.0427x; 1.7294x over previous
import functools

import jax
import jax.numpy as jnp
from jax import lax
from jax.experimental import pallas as pl
from jax.experimental.pallas import tpu as pltpu
from jax.experimental.pallas import tpu_sc as plsc

_N = 10000
_E = 320000
_NODE = 128
_HID = 32
_CH = 128
_NW = 32
_EPW = _E // _NW
_NFC = _EPW // _CH
_TAIL = _EPW - _NFC * _CH
_NBUF = 4
_NT = 16
_RPT = 624
_REM = _N - _NT * _RPT


def _sc_edge(eidx_hbm, a_hbm, p_hbm, q_hbm,
             s_out, d_out,
             sS, sD,
             ei0, ei1, ei2, ei3, a0, a1, a2, a3, p0, p1, p2, p3,
             q0, q1, q2, q3, ones_v, zbuf, dbuf, tidx,
             st0, st1, st2, st3, sg0, sg1, sg2, sg3, ss0, ss1, ss2, ss3):
    eiv = [ei0, ei1, ei2, ei3]
    av = [a0, a1, a2, a3]
    pv = [p0, p1, p2, p3]
    qv = [q0, q1, q2, q3]
    sst = [st0, st1, st2, st3]
    ssg = [sg0, sg1, sg2, sg3]
    sss = [ss0, ss1, ss2, ss3]
    cid = lax.axis_index("c")
    sid = lax.axis_index("s")
    wid = sid * 2 + cid
    wbase = wid * _EPW
    tb = sid * _RPT

    z16 = jnp.zeros((16,), jnp.float32)

    def _zrow(r, carry):
        zbuf[r, pl.ds(0, 16)] = z16
        zbuf[r, pl.ds(16, 16)] = z16
        return carry

    lax.fori_loop(0, _RPT, _zrow, 0, unroll=8)

    def _zdeg(k, carry):
        dbuf[pl.ds(k * 16, 16)] = z16
        return carry

    lax.fori_loop(0, _RPT // 16, _zdeg, 0)
    for k in range(_CH // 16):
        ones_v[pl.ds(k * 16, 16)] = jnp.full((16,), 1.0, jnp.float32)

    pltpu.sync_copy(zbuf, sS.at[pl.ds(tb, _RPT)])
    pltpu.sync_copy(dbuf, sD.at[pl.ds(tb, _RPT)])

    @pl.when(sid == _NT - 1)
    def _():
        pltpu.sync_copy(zbuf.at[pl.ds(0, _REM)],
                        sS.at[pl.ds(_NT * _RPT, _REM)])
        pltpu.sync_copy(dbuf.at[pl.ds(0, _REM)],
                        sD.at[pl.ds(_NT * _RPT, _REM)])

    plsc.subcore_barrier()

    wbase4 = wid * (_EPW // 4)

    def stage(ci, b):
        base = wbase + ci * _CH
        rbase = wbase4 + ci * (_CH // 4)
        pltpu.async_copy(eidx_hbm.at[:, pl.ds(base, _CH)], eiv[b], sst[b])
        pltpu.async_copy(a_hbm.at[pl.ds(rbase, _CH // 4)], av[b], sst[b])

    def wait_stage(b):
        pltpu.make_async_copy(eidx_hbm.at[:, pl.ds(0, _CH)], eiv[b],
                              sst[b]).wait()
        pltpu.make_async_copy(a_hbm.at[pl.ds(0, _CH // 4)], av[b],
                              sst[b]).wait()

    def gather(b):
        pltpu.async_copy(p_hbm.at[eiv[b].at[0]], pv[b], ssg[b])
        pltpu.async_copy(q_hbm.at[eiv[b].at[1]], qv[b], ssg[b])

    def wait_gather(b):
        pltpu.make_async_copy(p_hbm.at[eiv[b].at[0]], pv[b], ssg[b]).wait()
        pltpu.make_async_copy(q_hbm.at[eiv[b].at[1]], qv[b], ssg[b]).wait()

    def compute(b, nedges):
        def _grp(k, carry):
            for j in range(8):
                e = k * 8 + j
                ar = k * 2 + (j // 4)
                co = (j % 4) * 32
                pv[b][e, pl.ds(0, 16)] = jnp.maximum(
                    av[b][ar, pl.ds(co, 16)] + pv[b][e, pl.ds(0, 16)]
                    + qv[b][e, pl.ds(0, 16)], 0.0)
                pv[b][e, pl.ds(16, 16)] = jnp.maximum(
                    av[b][ar, pl.ds(co + 16, 16)] + pv[b][e, pl.ds(16, 16)]
                    + qv[b][e, pl.ds(16, 16)], 0.0)
            return carry

        lax.fori_loop(0, nedges // 8, _grp, 0)

    def scatter(b):
        pltpu.async_copy(pv[b], sS.at[eiv[b].at[0]], sss[b], add=True)
        pltpu.async_copy(ones_v, sD.at[eiv[b].at[0]], sss[b], add=True)

    def wait_scatter(b):
        pltpu.make_async_copy(pv[b], sS.at[eiv[b].at[0]], sss[b]).wait()
        pltpu.make_async_copy(ones_v, sD.at[eiv[b].at[0]], sss[b]).wait()

    stage(0, 0)
    stage(1, 1)
    stage(2, 2)
    wait_stage(0)
    gather(0)

    _MAIN = _NFC - 2 * _NBUF + 2

    def _main(g, carry):
        for r in range(_NBUF):
            i = g * _NBUF + r
            wait_stage((r + 1) % _NBUF)
            gather((r + 1) % _NBUF)
            if r == 0:
                @pl.when(g > 0)
                def _():
                    wait_scatter((r + 3) % _NBUF)
            else:
                wait_scatter((r + 3) % _NBUF)
            stage(i + 3, (r + 3) % _NBUF)
            wait_gather(r)
            compute(r, _CH)
            scatter(r)
        return carry

    lax.fori_loop(0, _MAIN // _NBUF, _main, 0)

    for i in range(_MAIN, _NFC):
        b = i % _NBUF
        if i + 1 < _NFC:
            wait_stage((i + 1) % _NBUF)
            gather((i + 1) % _NBUF)
        wait_scatter((i - 1) % _NBUF)
        if i + 3 < _NFC:
            stage(i + 3, (i + 3) % _NBUF)
        wait_gather(b)
        compute(b, _CH)
        scatter(b)
    wait_scatter((_NFC - 1) % _NBUF)

    tbase = wbase + _NFC * _CH
    pltpu.sync_copy(eidx_hbm.at[:, pl.ds(tbase, _TAIL)], tidx)
    pltpu.sync_copy(a_hbm.at[pl.ds(wbase4 + _NFC * (_CH // 4), _TAIL // 4)],
                    av[0].at[pl.ds(0, _TAIL // 4)])
    gp = pltpu.async_copy(p_hbm.at[tidx.at[0]], pv[0].at[pl.ds(0, _TAIL)],
                          sst[0])
    gq = pltpu.async_copy(q_hbm.at[tidx.at[1]], qv[0].at[pl.ds(0, _TAIL)],
                          sst[1])
    gp.wait()
    gq.wait()
    compute(0, _TAIL)
    pltpu.sync_copy(pv[0].at[pl.ds(0, _TAIL)], sS.at[tidx.at[0]], add=True)
    pltpu.sync_copy(ones_v.at[pl.ds(0, _TAIL)], sD.at[tidx.at[0]], add=True)

    plsc.subcore_barrier()

    pltpu.sync_copy(sS.at[pl.ds(tb, _RPT)], zbuf)
    pltpu.sync_copy(zbuf, s_out.at[cid, pl.ds(tb, _RPT)])
    pltpu.sync_copy(sD.at[pl.ds(tb, _RPT)], dbuf)
    pltpu.sync_copy(dbuf, d_out.at[cid, pl.ds(tb, _RPT)])

    @pl.when(sid == _NT - 1)
    def _():
        pltpu.sync_copy(sS.at[pl.ds(_NT * _RPT, _REM)],
                        zbuf.at[pl.ds(0, _REM)])
        pltpu.sync_copy(zbuf.at[pl.ds(0, _REM)],
                        s_out.at[cid, pl.ds(_NT * _RPT, _REM)])
        pltpu.sync_copy(sD.at[pl.ds(_NT * _RPT, _REM)],
                        dbuf.at[pl.ds(0, _REM)])
        pltpu.sync_copy(dbuf.at[pl.ds(0, _REM)],
                        d_out.at[cid, pl.ds(_NT * _RPT, _REM)])


def _pq_body(x_ref, wa_ref, wb_ref, p_ref, q_ref):
    xv = x_ref[...]
    p_ref[...] = jnp.dot(xv, wa_ref[...], preferred_element_type=jnp.float32)
    q_ref[...] = jnp.dot(xv, wb_ref[...], preferred_element_type=jnp.float32)


def _a_body(e_ref, w_ref, b_ref, o_ref):
    o_ref[...] = (jnp.dot(e_ref[...], w_ref[...],
                          preferred_element_type=jnp.float32) + b_ref[...])


def _head_body(x_ref, sp_ref, dp_ref, w2_ref, b2_ref,
               wcx_ref, wca_ref, wmx_ref, wma_ref, wsx_ref, wsa_ref,
               bc_ref, bm_ref, bs_ref, hi_ref, inv_ref, ord_ref):
    s = sp_ref[0] + sp_ref[1]
    deg = dp_ref[0] + dp_ref[1]
    agg = (jnp.dot(s, w2_ref[...], preferred_element_type=jnp.float32)
           + deg * b2_ref[...])
    xv = x_ref[...]
    cx = (jnp.dot(xv, wcx_ref[...], preferred_element_type=jnp.float32)
          + jnp.dot(agg, wca_ref[...], preferred_element_type=jnp.float32)
          + bc_ref[...] + 1e-10)
    conc = jax.nn.softplus(cx)
    inv_ref[...] = conc / (jnp.sum(conc) + 1e-20)
    am = (jnp.dot(xv, wmx_ref[...], preferred_element_type=jnp.float32)
          + jnp.dot(agg, wma_ref[...], preferred_element_type=jnp.float32)
          + bm_ref[...])
    av = (jnp.dot(xv, wsx_ref[...], preferred_element_type=jnp.float32)
          + jnp.dot(agg, wsa_ref[...], preferred_element_type=jnp.float32)
          + bs_ref[...])
    alpha = jax.nn.softplus(am + 1e-20) + 1e-20
    beta = jax.nn.softplus(av + 1e-20) + 1.0
    ord_ref[...] = alpha / (alpha + beta) * hi_ref[0, 0]


def kernel(x, edge_index, edge_attr, W1, b1, W2, b2, Wc, bc, Wm, bm, Ws, bs,
           high, deterministic):
    f32 = jnp.float32
    w1a = W1[:_NODE]
    w1b = W1[_NODE:2 * _NODE]
    w1c = W1[2 * _NODE:]

    p_arr, q_arr = pl.pallas_call(
        _pq_body,
        out_shape=[jax.ShapeDtypeStruct((_N, _HID), f32)] * 2,
    )(x, w1a, w1b)

    ea4 = edge_attr.reshape(_E // 4, 64)
    w1c4 = jnp.zeros((64, 128), f32)
    for k in range(4):
        w1c4 = w1c4.at[16 * k:16 * (k + 1), _HID * k:_HID * (k + 1)].set(w1c)
    b14 = jnp.tile(b1, 4).reshape(1, 128)
    er = _E // 4 // 10
    a_arr = pl.pallas_call(
        _a_body,
        grid=(10,),
        in_specs=[pl.BlockSpec((er, 64), lambda i: (i, 0)),
                  pl.BlockSpec((64, 128), lambda i: (0, 0)),
                  pl.BlockSpec((1, 128), lambda i: (0, 0))],
        out_specs=pl.BlockSpec((er, 128), lambda i: (i, 0)),
        out_shape=jax.ShapeDtypeStruct((_E // 4, 128), f32),
    )(ea4, w1c4, b14)

    sc_fn = pl.kernel(
        _sc_edge,
        out_type=[jax.ShapeDtypeStruct((2, _N, _HID), f32),
                  jax.ShapeDtypeStruct((2, _N), f32)],
        mesh=plsc.VectorSubcoreMesh(core_axis_name="c", subcore_axis_name="s"),
        compiler_params=pltpu.CompilerParams(use_tc_tiling_on_sc=False),
        scratch_types=(
            [pltpu.VMEM_SHARED((_N, _HID), f32),
             pltpu.VMEM_SHARED((_N,), f32)]
            + [pltpu.VMEM((2, _CH), jnp.int32)] * _NBUF
            + [pltpu.VMEM((_CH // 4, 128), f32)] * _NBUF
            + [pltpu.VMEM((_CH, _HID), f32)] * (2 * _NBUF)
            + [pltpu.VMEM((_CH,), f32),
               pltpu.VMEM((_RPT, _HID), f32),
               pltpu.VMEM((_RPT,), f32),
               pltpu.VMEM((2, _TAIL), jnp.int32)]
            + [pltpu.SemaphoreType.DMA] * (3 * _NBUF)
        ),
    )
    s_part, d_part = sc_fn(edge_index, a_arr, p_arr, q_arr)
    d_part = d_part.reshape(2, _N, 1)

    inv, ordv = pl.pallas_call(
        _head_body,
        out_shape=[jax.ShapeDtypeStruct((_N, 1), f32)] * 2,
    )(x, s_part, d_part, W2, b2.reshape(1, _HID),
      Wc[:_NODE], Wc[_NODE:], Wm[:_NODE], Wm[_NODE:], Ws[:_NODE], Ws[_NODE:],
      bc.reshape(1, 1), bm.reshape(1, 1), bs.reshape(1, 1),
      jnp.reshape(high, (1, 1)).astype(f32))

    inventory_act = inv.reshape(100, 100)
    order_act = ordv.reshape(100, 100)[:, -10:].reshape(-1)
    return (inventory_act, order_act)


# speedup vs baseline: 9.2841x; 1.0267x over previous
import functools

import jax
import jax.numpy as jnp
from jax import lax
from jax.experimental import pallas as pl
from jax.experimental.pallas import tpu as pltpu
from jax.experimental.pallas import tpu_sc as plsc

_N = 10000
_E = 320000
_NODE = 128
_HID = 32
_CH = 128
_NW = 32
_NCHK = _E // _CH
_NFC = _NCHK // _NW
_NXTRA = _NCHK - _NFC * _NW
_NBUF = 4
_NT = 16
_RPT = 624
_REM = _N - _NT * _RPT


def _sc_edge(eidx_hbm, a_hbm, p_hbm, q_hbm,
             s_out, d_out,
             sS, sD,
             ei0, ei1, ei2, ei3, a0, a1, a2, a3, p0, p1, p2, p3,
             q0, q1, q2, q3, ones_v, zbuf, dbuf,
             st0, st1, st2, st3, sg0, sg1, sg2, sg3, ss0, ss1, ss2, ss3):
    eiv = [ei0, ei1, ei2, ei3]
    av = [a0, a1, a2, a3]
    pv = [p0, p1, p2, p3]
    qv = [q0, q1, q2, q3]
    sst = [st0, st1, st2, st3]
    ssg = [sg0, sg1, sg2, sg3]
    sss = [ss0, ss1, ss2, ss3]
    cid = lax.axis_index("c")
    sid = lax.axis_index("s")
    wid = sid * 2 + cid
    wchunk = wid * _NFC
    tb = sid * _RPT

    z16 = jnp.zeros((16,), jnp.float32)

    def _zrow(r, carry):
        zbuf[r, pl.ds(0, 16)] = z16
        zbuf[r, pl.ds(16, 16)] = z16
        return carry

    lax.fori_loop(0, _RPT, _zrow, 0, unroll=8)

    def _zdeg(k, carry):
        dbuf[pl.ds(k * 16, 16)] = z16
        return carry

    lax.fori_loop(0, _RPT // 16, _zdeg, 0)
    for k in range(_CH // 16):
        ones_v[pl.ds(k * 16, 16)] = jnp.full((16,), 1.0, jnp.float32)

    pltpu.sync_copy(zbuf, sS.at[pl.ds(tb, _RPT)])
    pltpu.sync_copy(dbuf, sD.at[pl.ds(tb, _RPT)])

    @pl.when(sid == _NT - 1)
    def _():
        pltpu.sync_copy(zbuf.at[pl.ds(0, _REM)],
                        sS.at[pl.ds(_NT * _RPT, _REM)])
        pltpu.sync_copy(dbuf.at[pl.ds(0, _REM)],
                        sD.at[pl.ds(_NT * _RPT, _REM)])

    plsc.subcore_barrier()

    def stage(ci, b):
        g = wchunk + ci
        pltpu.async_copy(eidx_hbm.at[:, pl.ds(g * _CH, _CH)], eiv[b], sst[b])
        pltpu.async_copy(a_hbm.at[g], av[b], sst[b])

    def wait_stage(b):
        pltpu.make_async_copy(eidx_hbm.at[:, pl.ds(0, _CH)], eiv[b],
                              sst[b]).wait()
        pltpu.make_async_copy(a_hbm.at[0], av[b], sst[b]).wait()

    def gather(b):
        pltpu.async_copy(p_hbm.at[eiv[b].at[0]], pv[b], ssg[b])
        pltpu.async_copy(q_hbm.at[eiv[b].at[1]], qv[b], ssg[b])

    def wait_gather(b):
        pltpu.make_async_copy(p_hbm.at[eiv[b].at[0]], pv[b], ssg[b]).wait()
        pltpu.make_async_copy(q_hbm.at[eiv[b].at[1]], qv[b], ssg[b]).wait()

    def compute(b, nedges):
        def _grp(k, carry):
            for j in range(8):
                e = k * 8 + j
                ar = k * 2 + (j // 4)
                co = (j % 4) * 32
                pv[b][e, pl.ds(0, 16)] = jnp.maximum(
                    av[b][ar, pl.ds(co, 16)] + pv[b][e, pl.ds(0, 16)]
                    + qv[b][e, pl.ds(0, 16)], 0.0)
                pv[b][e, pl.ds(16, 16)] = jnp.maximum(
                    av[b][ar, pl.ds(co + 16, 16)] + pv[b][e, pl.ds(16, 16)]
                    + qv[b][e, pl.ds(16, 16)], 0.0)
            return carry

        lax.fori_loop(0, nedges // 8, _grp, 0)

    def scatter(b):
        pltpu.async_copy(pv[b], sS.at[eiv[b].at[0]], sss[b], add=True)
        pltpu.async_copy(ones_v, sD.at[eiv[b].at[0]], sss[b], add=True)

    def wait_scatter(b):
        pltpu.make_async_copy(pv[b], sS.at[eiv[b].at[0]], sss[b]).wait()
        pltpu.make_async_copy(ones_v, sD.at[eiv[b].at[0]], sss[b]).wait()

    stage(0, 0)
    stage(1, 1)
    stage(2, 2)
    wait_stage(0)
    gather(0)

    _MAIN = _NFC - 2 * _NBUF + 2

    def _main(g, carry):
        for r in range(_NBUF):
            i = g * _NBUF + r
            wait_stage((r + 1) % _NBUF)
            gather((r + 1) % _NBUF)
            if r == 0:
                @pl.when(g > 0)
                def _():
                    wait_scatter((r + 3) % _NBUF)
            else:
                wait_scatter((r + 3) % _NBUF)
            stage(i + 3, (r + 3) % _NBUF)
            wait_gather(r)
            compute(r, _CH)
            scatter(r)
        return carry

    lax.fori_loop(0, _MAIN // _NBUF, _main, 0)

    for i in range(_MAIN, _NFC):
        b = i % _NBUF
        if i + 1 < _NFC:
            wait_stage((i + 1) % _NBUF)
            gather((i + 1) % _NBUF)
        wait_scatter((i - 1) % _NBUF)
        if i + 3 < _NFC:
            stage(i + 3, (i + 3) % _NBUF)
        wait_gather(b)
        compute(b, _CH)
        scatter(b)
    wait_scatter((_NFC - 1) % _NBUF)

    @pl.when(wid < _NXTRA)
    def _():
        g = _NW * _NFC + wid
        pltpu.sync_copy(eidx_hbm.at[:, pl.ds(g * _CH, _CH)], eiv[0])
        pltpu.sync_copy(a_hbm.at[g], av[0])
        gp = pltpu.async_copy(p_hbm.at[eiv[0].at[0]], pv[0], sst[0])
        gq = pltpu.async_copy(q_hbm.at[eiv[0].at[1]], qv[0], sst[1])
        gp.wait()
        gq.wait()
        compute(0, _CH)
        pltpu.sync_copy(pv[0], sS.at[eiv[0].at[0]], add=True)
        pltpu.sync_copy(ones_v, sD.at[eiv[0].at[0]], add=True)

    plsc.subcore_barrier()

    pltpu.sync_copy(sS.at[pl.ds(tb, _RPT)], zbuf)
    pltpu.sync_copy(zbuf, s_out.at[cid, pl.ds(tb, _RPT)])
    pltpu.sync_copy(sD.at[pl.ds(tb, _RPT)], dbuf)
    pltpu.sync_copy(dbuf, d_out.at[cid, pl.ds(tb, _RPT)])

    @pl.when(sid == _NT - 1)
    def _():
        pltpu.sync_copy(sS.at[pl.ds(_NT * _RPT, _REM)],
                        zbuf.at[pl.ds(0, _REM)])
        pltpu.sync_copy(zbuf.at[pl.ds(0, _REM)],
                        s_out.at[cid, pl.ds(_NT * _RPT, _REM)])
        pltpu.sync_copy(sD.at[pl.ds(_NT * _RPT, _REM)],
                        dbuf.at[pl.ds(0, _REM)])
        pltpu.sync_copy(dbuf.at[pl.ds(0, _REM)],
                        d_out.at[cid, pl.ds(_NT * _RPT, _REM)])


def _pq_body(x_ref, wa_ref, wb_ref, p_ref, q_ref):
    xv = x_ref[...]
    p_ref[...] = jnp.dot(xv, wa_ref[...], preferred_element_type=jnp.float32)
    q_ref[...] = jnp.dot(xv, wb_ref[...], preferred_element_type=jnp.float32)


def _a_body(e_ref, w_ref, b_ref, o_ref):
    y = (jnp.dot(e_ref[...], w_ref[...],
                 preferred_element_type=jnp.float32) + b_ref[...])
    o_ref[...] = y.reshape(o_ref.shape)


def _head_body(x_ref, sp_ref, dp_ref, w2_ref, b2_ref,
               wcx_ref, wca_ref, wmx_ref, wma_ref, wsx_ref, wsa_ref,
               bc_ref, bm_ref, bs_ref, hi_ref, inv_ref, ord_ref):
    s = sp_ref[0] + sp_ref[1]
    deg = dp_ref[0] + dp_ref[1]
    agg = (jnp.dot(s, w2_ref[...], preferred_element_type=jnp.float32)
           + deg * b2_ref[...])
    xv = x_ref[...]
    cx = (jnp.dot(xv, wcx_ref[...], preferred_element_type=jnp.float32)
          + jnp.dot(agg, wca_ref[...], preferred_element_type=jnp.float32)
          + bc_ref[...] + 1e-10)
    conc = jax.nn.softplus(cx)
    inv_ref[...] = conc / (jnp.sum(conc) + 1e-20)
    am = (jnp.dot(xv, wmx_ref[...], preferred_element_type=jnp.float32)
          + jnp.dot(agg, wma_ref[...], preferred_element_type=jnp.float32)
          + bm_ref[...])
    av = (jnp.dot(xv, wsx_ref[...], preferred_element_type=jnp.float32)
          + jnp.dot(agg, wsa_ref[...], preferred_element_type=jnp.float32)
          + bs_ref[...])
    alpha = jax.nn.softplus(am + 1e-20) + 1e-20
    beta = jax.nn.softplus(av + 1e-20) + 1.0
    ord_ref[...] = alpha / (alpha + beta) * hi_ref[0, 0]


def kernel(x, edge_index, edge_attr, W1, b1, W2, b2, Wc, bc, Wm, bm, Ws, bs,
           high, deterministic):
    f32 = jnp.float32
    w1a = W1[:_NODE]
    w1b = W1[_NODE:2 * _NODE]
    w1c = W1[2 * _NODE:]

    p_arr, q_arr = pl.pallas_call(
        _pq_body,
        out_shape=[jax.ShapeDtypeStruct((_N, _HID), f32)] * 2,
    )(x, w1a, w1b)

    ea8 = edge_attr.reshape(_E // 8, 128)
    w1c8 = jnp.zeros((128, 256), f32)
    for k in range(8):
        w1c8 = w1c8.at[16 * k:16 * (k + 1), _HID * k:_HID * (k + 1)].set(w1c)
    b18 = jnp.tile(b1, 8).reshape(1, 256)
    er = _E // 8 // 10
    a_arr = pl.pallas_call(
        _a_body,
        grid=(10,),
        in_specs=[pl.BlockSpec((er, 128), lambda i: (i, 0)),
                  pl.BlockSpec((128, 256), lambda i: (0, 0)),
                  pl.BlockSpec((1, 256), lambda i: (0, 0))],
        out_specs=pl.BlockSpec((er // 16, 32, 128), lambda i: (i, 0, 0)),
        out_shape=jax.ShapeDtypeStruct((_E // _CH, 32, 128), f32),
    )(ea8, w1c8, b18)

    sc_fn = pl.kernel(
        _sc_edge,
        out_type=[jax.ShapeDtypeStruct((2, _N, _HID), f32),
                  jax.ShapeDtypeStruct((2, _N), f32)],
        mesh=plsc.VectorSubcoreMesh(core_axis_name="c", subcore_axis_name="s"),
        compiler_params=pltpu.CompilerParams(use_tc_tiling_on_sc=False),
        scratch_types=(
            [pltpu.VMEM_SHARED((_N, _HID), f32),
             pltpu.VMEM_SHARED((_N,), f32)]
            + [pltpu.VMEM((2, _CH), jnp.int32)] * _NBUF
            + [pltpu.VMEM((_CH // 4, 128), f32)] * _NBUF
            + [pltpu.VMEM((_CH, _HID), f32)] * (2 * _NBUF)
            + [pltpu.VMEM((_CH,), f32),
               pltpu.VMEM((_RPT, _HID), f32),
               pltpu.VMEM((_RPT,), f32)]
            + [pltpu.SemaphoreType.DMA] * (3 * _NBUF)
        ),
    )
    s_part, d_part = sc_fn(edge_index, a_arr, p_arr, q_arr)
    d_part = d_part.reshape(2, _N, 1)

    inv, ordv = pl.pallas_call(
        _head_body,
        out_shape=[jax.ShapeDtypeStruct((_N, 1), f32)] * 2,
    )(x, s_part, d_part, W2, b2.reshape(1, _HID),
      Wc[:_NODE], Wc[_NODE:], Wm[:_NODE], Wm[_NODE:], Ws[:_NODE], Ws[_NODE:],
      bc.reshape(1, 1), bm.reshape(1, 1), bs.reshape(1, 1),
      jnp.reshape(high, (1, 1)).astype(f32))

    inventory_act = inv.reshape(100, 100)
    order_act = ordv.reshape(100, 100)[:, -10:].reshape(-1)
    return (inventory_act, order_act)
